# Initial kernel scaffold; baseline (speedup 1.0000x reference)
import jax
import jax.numpy as jnp
from jax.experimental import pallas as pl


def kernel(sequence_tensor, masked_lm_positions):
    raise NotImplementedError("write your pallas kernel here")


# speedup vs baseline: 1.2655x; 1.2655x over previous
import functools

import jax
import jax.numpy as jnp
from jax import lax
from jax.experimental import pallas as pl
from jax.experimental.pallas import tpu as pltpu
from jax.experimental.pallas import tpu_sc as plsc


@functools.partial(jax.jit, static_argnums=(2, 3, 4, 5))
def _gather_rows(table, idx, B, P, S, D):
    info = plsc.get_sparse_core_info()
    NC, NS, L = info.num_cores, info.num_subcores, info.num_lanes
    NW = NC * NS
    N = B * P
    assert N % NW == 0
    b_per_w = N // NW
    assert b_per_w % L == 0 and (b_per_w * D * 4) <= 500_000

    mesh = plsc.VectorSubcoreMesh(core_axis_name="c", subcore_axis_name="s")

    @functools.partial(
        pl.kernel,
        mesh=mesh,
        out_type=jax.ShapeDtypeStruct((N, D), jnp.float32),
        scratch_types=[
            pltpu.VMEM((b_per_w,), jnp.int32),
            pltpu.VMEM((b_per_w, D), jnp.float32),
            pltpu.SemaphoreType.DMA,
        ],
    )
    def k(table_hbm, idx_hbm, out_hbm, idx_v, rows_v, sem):
        wid = lax.axis_index("s") * NC + lax.axis_index("c")
        base = wid * b_per_w
        pltpu.sync_copy(idx_hbm.at[pl.ds(base, b_per_w)], idx_v)
        off = (base // P) * S
        for i in range(b_per_w // L):
            idx_v[pl.ds(i * L, L)] = idx_v[pl.ds(i * L, L)] + off
        pltpu.async_copy(table_hbm.at[idx_v], rows_v, sem).wait()
        pltpu.sync_copy(rows_v, out_hbm.at[pl.ds(base, b_per_w)])

    return k(table, idx)


def kernel(sequence_tensor, masked_lm_positions):
    B, S, D = sequence_tensor.shape
    _, P = masked_lm_positions.shape
    table = sequence_tensor.reshape(B * S, D)
    idx = masked_lm_positions.astype(jnp.int32).reshape(B * P)
    out = _gather_rows(table, idx, B, P, S, D)
    return out.reshape(B, P, D)
